# Initial kernel scaffold; baseline (speedup 1.0000x reference)
import functools

import jax
import jax.numpy as jnp
from jax import lax
from jax.experimental import pallas as pl
from jax.experimental.pallas import tpu as pltpu
from jax.experimental.pallas import tpu_sc as plsc

VOCAB_SIZE = 32000
N_EMBD = 1024
B, T = 4, 2048

NC, NS, L = 2, 16, 16
NW = NC * NS
T_PER_W = T // NW
CHUNK = 32
N_TCHUNK = T_PER_W // CHUNK
VPR = N_EMBD // L


def _body(idx_hbm, pos_hbm, tok_hbm, out_hbm, idx_v, pos_v, tok_v, sem):
    w = lax.axis_index("s") * NC + lax.axis_index("c")
    t0 = w * T_PER_W

    for b in range(B):
        pltpu.sync_copy(
            idx_hbm.at[pl.ds(b * T + t0, T_PER_W)],
            idx_v.at[pl.ds(b * T_PER_W, T_PER_W)],
        )

    for tc in range(N_TCHUNK):
        pltpu.sync_copy(pos_hbm.at[pl.ds(t0 + tc * CHUNK, CHUNK)], pos_v)
        for b in range(B):
            pltpu.async_copy(
                tok_hbm.at[idx_v.at[pl.ds(b * T_PER_W + tc * CHUNK, CHUNK)]],
                tok_v,
                sem,
            ).wait()

            def add_row(r, carry):
                for j in range(VPR):
                    sl = pl.ds(j * L, L)
                    plsc.addupdate(tok_v.at[r, sl], pos_v[r, sl])
                return carry

            lax.fori_loop(0, CHUNK, add_row, 0)

            pltpu.sync_copy(
                tok_v,
                out_hbm.at[pl.ds(b * T + t0 + tc * CHUNK, CHUNK)],
            )


@jax.jit
def kernel(idx, tok_table, pos_table):
    idx_flat = idx.reshape(B * T).astype(jnp.int32)
    mesh = plsc.VectorSubcoreMesh(core_axis_name="c", subcore_axis_name="s")
    out = pl.kernel(
        _body,
        out_type=jax.ShapeDtypeStruct((B * T, N_EMBD), jnp.float32),
        mesh=mesh,
        scratch_types=[
            pltpu.VMEM((B * T_PER_W,), jnp.int32),
            pltpu.VMEM((CHUNK, N_EMBD), jnp.float32),
            pltpu.VMEM((CHUNK, N_EMBD), jnp.float32),
            pltpu.SemaphoreType.DMA,
        ],
    )(idx_flat, pos_table, tok_table)
    return out.reshape(B, T, N_EMBD)


# speedup vs baseline: 1.0307x; 1.0307x over previous
import functools

import jax
import jax.numpy as jnp
from jax import lax
from jax.experimental import pallas as pl
from jax.experimental.pallas import tpu as pltpu
from jax.experimental.pallas import tpu_sc as plsc

VOCAB_SIZE = 32000
N_EMBD = 1024
B, T = 4, 2048

NC, NS, L = 2, 16, 16
NW = NC * NS
T_PER_W = T // NW
CHUNK = 32
N_TCHUNK = T_PER_W // CHUNK
VPR = N_EMBD // L


def _body(idx_hbm, pos_hbm, tok_hbm, out_hbm, idx_v, pos_v, tok_v, sem):
    w = lax.axis_index("s") * NC + lax.axis_index("c")
    t0 = w * T_PER_W

    for b in range(B):
        pltpu.sync_copy(
            idx_hbm.at[pl.ds(b * T + t0, T_PER_W)],
            idx_v.at[pl.ds(b * T_PER_W, T_PER_W)],
        )

    for tc in range(N_TCHUNK):
        pltpu.sync_copy(pos_hbm.at[pl.ds(t0 + tc * CHUNK, CHUNK)], pos_v)
        for b in range(B):
            pltpu.async_copy(
                tok_hbm.at[idx_v.at[pl.ds(b * T_PER_W + tc * CHUNK, CHUNK)]],
                tok_v,
                sem,
            ).wait()

            def add_row(r, carry):
                for j in range(VPR):
                    sl = pl.ds(j * L, L)
                    plsc.addupdate(tok_v.at[r, sl], pos_v[r, sl])
                return carry

            lax.fori_loop(0, CHUNK, add_row, 0)

            pltpu.sync_copy(
                tok_v,
                out_hbm.at[pl.ds(b * T + t0 + tc * CHUNK, CHUNK)],
            )


@jax.jit
def kernel(idx, tok_table, pos_table):
    idx_flat = idx.reshape(B * T).astype(jnp.int32)
    mesh = plsc.VectorSubcoreMesh(
        core_axis_name="c", subcore_axis_name="s", num_cores=NC, num_subcores=NS
    )
    out = pl.kernel(
        _body,
        out_type=jax.ShapeDtypeStruct((B * T, N_EMBD), jnp.float32),
        mesh=mesh,
        scratch_types=[
            pltpu.VMEM((B * T_PER_W,), jnp.int32),
            pltpu.VMEM((CHUNK, N_EMBD), jnp.float32),
            pltpu.VMEM((CHUNK, N_EMBD), jnp.float32),
            pltpu.SemaphoreType.DMA,
        ],
    )(idx_flat, pos_table, tok_table)
    return out.reshape(B, T, N_EMBD)


# speedup vs baseline: 1.0973x; 1.0646x over previous
import jax
import jax.numpy as jnp
from jax import lax
from jax.experimental import pallas as pl
from jax.experimental.pallas import tpu as pltpu
from jax.experimental.pallas import tpu_sc as plsc

VOCAB_SIZE = 32000
N_EMBD = 1024
B, T = 4, 2048

NC, NS, L = 2, 16, 16
NW = NC * NS
T_PER_W = T // NW
CHUNK = 16
N_TCHUNK = T_PER_W // CHUNK
NSTEP = N_TCHUNK * B
NBUF = 3
VPR = N_EMBD // L


def _step_slices(s, t0):
    tc, b = divmod(s, B)
    return (
        b * T_PER_W + tc * CHUNK,
        b * T + t0 + tc * CHUNK,
        tc * CHUNK,
    )


def _body(idx_hbm, pos_hbm, tok_hbm, out_hbm, idx_v, pos_v, toks, gsems, ssems,
          psem):
    w = lax.axis_index("s") * NC + lax.axis_index("c")
    t0 = w * T_PER_W

    pos_cp = pltpu.async_copy(pos_hbm.at[pl.ds(t0, T_PER_W)], pos_v, psem)

    for b in range(B):
        pltpu.sync_copy(
            idx_hbm.at[pl.ds(b * T + t0, T_PER_W)],
            idx_v.at[pl.ds(b * T_PER_W, T_PER_W)],
        )

    def fire_gather(s, j):
        off, _, _ = _step_slices(s, t0)
        pltpu.async_copy(
            tok_hbm.at[idx_v.at[pl.ds(off, CHUNK)]], toks[j], gsems[j])

    for j in range(NBUF):
        fire_gather(j, j)
    pos_cp.wait()

    for s in range(NSTEP):
        j = s % NBUF
        off, out_row, pos_row = _step_slices(s, t0)
        pltpu.make_async_copy(
            tok_hbm.at[idx_v.at[pl.ds(off, CHUNK)]], toks[j], gsems[j]).wait()

        def add_row(r, carry, j=j, pos_row=pos_row):
            for v in range(VPR):
                sl = pl.ds(v * L, L)
                plsc.addupdate(toks[j].at[r, sl], pos_v[pos_row + r, sl])
            return carry

        lax.fori_loop(0, CHUNK, add_row, 0)

        scat = pltpu.async_copy(
            toks[j], out_hbm.at[pl.ds(out_row, CHUNK)], ssems[j])
        if s + NBUF < NSTEP:
            scat.wait()
            fire_gather(s + NBUF, j)
        else:
            scat.wait()


@jax.jit
def kernel(idx, tok_table, pos_table):
    idx_flat = idx.reshape(B * T).astype(jnp.int32)
    mesh = plsc.VectorSubcoreMesh(
        core_axis_name="c", subcore_axis_name="s", num_cores=NC,
        num_subcores=NS)

    def body(idx_hbm, pos_hbm, tok_hbm, out_hbm, idx_v, pos_v,
             t0, t1, t2, g0, g1, g2, s0, s1, s2, psem):
        _body(idx_hbm, pos_hbm, tok_hbm, out_hbm, idx_v, pos_v,
              [t0, t1, t2], [g0, g1, g2], [s0, s1, s2], psem)

    out = pl.kernel(
        body,
        out_type=jax.ShapeDtypeStruct((B * T, N_EMBD), jnp.float32),
        mesh=mesh,
        scratch_types=[
            pltpu.VMEM((B * T_PER_W,), jnp.int32),
            pltpu.VMEM((T_PER_W, N_EMBD), jnp.float32),
        ] + [pltpu.VMEM((CHUNK, N_EMBD), jnp.float32)] * NBUF
          + [pltpu.SemaphoreType.DMA] * (2 * NBUF + 1),
    )(idx_flat, pos_table, tok_table)
    return out.reshape(B, T, N_EMBD)


# speedup vs baseline: 1.4192x; 1.2933x over previous
import jax
import jax.numpy as jnp
from jax import lax
from jax.experimental import pallas as pl
from jax.experimental.pallas import tpu as pltpu
from jax.experimental.pallas import tpu_sc as plsc

VOCAB_SIZE = 32000
N_EMBD = 1024
B, T = 4, 2048

NC, NS, L = 2, 16, 16
NW = NC * NS
T_PER_W = T // NW
CHUNK = 16
N_TCHUNK = T_PER_W // CHUNK
NSTEP = N_TCHUNK * B
NBUF = 4
LA = 2
VPR = N_EMBD // L


def _step_slices(s, t0):
    tc, b = divmod(s, B)
    return (
        b * T_PER_W + tc * CHUNK,
        b * T + t0 + tc * CHUNK,
        tc,
    )


def _body(idx_hbm, pos_hbm, tok_hbm, out_hbm, idx_v, poss, toks, gsems, ssems,
          psems):
    w = lax.axis_index("s") * NC + lax.axis_index("c")
    t0 = w * T_PER_W

    def fire_pos(tc):
        pltpu.async_copy(
            pos_hbm.at[pl.ds(t0 + tc * CHUNK, CHUNK)],
            poss[tc % 2],
            psems[tc % 2],
        )

    def wait_pos(tc):
        pltpu.make_async_copy(
            pos_hbm.at[pl.ds(t0 + tc * CHUNK, CHUNK)],
            poss[tc % 2],
            psems[tc % 2],
        ).wait()

    fire_pos(0)
    fire_pos(1)

    for b in range(B):
        pltpu.sync_copy(
            idx_hbm.at[pl.ds(b * T + t0, T_PER_W)],
            idx_v.at[pl.ds(b * T_PER_W, T_PER_W)],
        )

    def fire_gather(s):
        off, _, _ = _step_slices(s, t0)
        j = s % NBUF
        pltpu.async_copy(
            tok_hbm.at[idx_v.at[pl.ds(off, CHUNK)]], toks[j], gsems[j])

    def wait_gather(s):
        off, _, _ = _step_slices(s, t0)
        j = s % NBUF
        pltpu.make_async_copy(
            tok_hbm.at[idx_v.at[pl.ds(off, CHUNK)]], toks[j], gsems[j]).wait()

    def wait_scatter(s):
        _, out_row, _ = _step_slices(s, t0)
        j = s % NBUF
        pltpu.make_async_copy(
            toks[j], out_hbm.at[pl.ds(out_row, CHUNK)], ssems[j]).wait()

    for s in range(LA):
        fire_gather(s)

    for s in range(NSTEP):
        j = s % NBUF
        _, out_row, tc = _step_slices(s, t0)
        wait_gather(s)
        if s % B == 0:
            wait_pos(tc)
            if 1 <= tc < N_TCHUNK - 1:
                fire_pos(tc + 1)

        pv = poss[tc % 2]

        def add_row(r, carry, j=j, pv=pv):
            for v in range(VPR):
                sl = pl.ds(v * L, L)
                plsc.addupdate(toks[j].at[r, sl], pv[r, sl])
            return carry

        lax.fori_loop(0, CHUNK, add_row, 0)

        pltpu.async_copy(toks[j], out_hbm.at[pl.ds(out_row, CHUNK)], ssems[j])

        s2 = s + LA
        if s2 < NSTEP:
            if s2 >= NBUF:
                wait_scatter(s2 - NBUF)
            fire_gather(s2)

    for s in range(NSTEP - NBUF, NSTEP):
        wait_scatter(s)


@jax.jit
def kernel(idx, tok_table, pos_table):
    idx_flat = idx.reshape(B * T).astype(jnp.int32)
    mesh = plsc.VectorSubcoreMesh(
        core_axis_name="c", subcore_axis_name="s", num_cores=NC,
        num_subcores=NS)

    def body(idx_hbm, pos_hbm, tok_hbm, out_hbm, idx_v, p0, p1,
             t0, t1, t2, t3, g0, g1, g2, g3, s0, s1, s2, s3, ps0, ps1):
        _body(idx_hbm, pos_hbm, tok_hbm, out_hbm, idx_v, [p0, p1],
              [t0, t1, t2, t3], [g0, g1, g2, g3], [s0, s1, s2, s3],
              [ps0, ps1])

    out = pl.kernel(
        body,
        out_type=jax.ShapeDtypeStruct((B * T, N_EMBD), jnp.float32),
        mesh=mesh,
        scratch_types=[
            pltpu.VMEM((B * T_PER_W,), jnp.int32),
        ] + [pltpu.VMEM((CHUNK, N_EMBD), jnp.float32)] * 2
          + [pltpu.VMEM((CHUNK, N_EMBD), jnp.float32)] * NBUF
          + [pltpu.SemaphoreType.DMA] * (2 * NBUF + 2),
    )(idx_flat, pos_table, tok_table)
    return out.reshape(B, T, N_EMBD)


# speedup vs baseline: 1.4391x; 1.0140x over previous
import jax
import jax.numpy as jnp
from jax import lax
from jax.experimental import pallas as pl
from jax.experimental.pallas import tpu as pltpu
from jax.experimental.pallas import tpu_sc as plsc

VOCAB_SIZE = 32000
N_EMBD = 1024
B, T = 4, 2048

NC, NS, L = 2, 16, 16
NW = NC * NS
T_PER_W = T // NW
CHUNK = 16
N_TCHUNK = T_PER_W // CHUNK
NSTEP = N_TCHUNK * B
NBUF = 4
LA = 2
VPR = N_EMBD // L


def _body(idx_hbm, pos_hbm, tok_hbm, out_hbm, idx_v, poss, toks, gsems, ssems,
          psems, isem):
    w = lax.axis_index("s") * NC + lax.axis_index("c")
    t0 = w * T_PER_W

    def fire_pos(tc):
        pltpu.async_copy(
            pos_hbm.at[pl.ds(t0 + tc * CHUNK, CHUNK)],
            poss[tc % 2],
            psems[tc % 2],
        )

    def wait_pos(tc):
        pltpu.make_async_copy(
            pos_hbm.at[pl.ds(t0 + tc * CHUNK, CHUNK)],
            poss[tc % 2],
            psems[tc % 2],
        ).wait()

    fire_pos(0)
    fire_pos(1)

    for b in range(B):
        pltpu.async_copy(
            idx_hbm.at[b, pl.ds(t0, T_PER_W)],
            idx_v.at[pl.ds(b * T_PER_W, T_PER_W)], isem)
    for b in range(B):
        pltpu.make_async_copy(
            idx_hbm.at[b, pl.ds(t0, T_PER_W)],
            idx_v.at[pl.ds(b * T_PER_W, T_PER_W)], isem).wait()

    def _idx_ref(s):
        tc, b = divmod(s, B)
        return idx_v.at[pl.ds(b * T_PER_W + tc * CHUNK, CHUNK)]

    def fire_gather(s):
        j = s % NBUF
        pltpu.async_copy(tok_hbm.at[_idx_ref(s)], toks[j], gsems[j])

    def wait_gather(s):
        j = s % NBUF
        pltpu.make_async_copy(tok_hbm.at[_idx_ref(s)], toks[j], gsems[j]).wait()

    def fire_scatter(s):
        tc, b = divmod(s, B)
        j = s % NBUF
        pltpu.async_copy(
            toks[j], out_hbm.at[b, pl.ds(t0 + tc * CHUNK, CHUNK)], ssems[j])

    def wait_scatter(s):
        tc, b = divmod(s, B)
        j = s % NBUF
        pltpu.make_async_copy(
            toks[j], out_hbm.at[b, pl.ds(t0 + tc * CHUNK, CHUNK)],
            ssems[j]).wait()

    for s in range(LA):
        fire_gather(s)

    for s in range(NSTEP):
        j = s % NBUF
        tc = s // B
        wait_gather(s)
        if s % B == 0:
            wait_pos(tc)
            if 1 <= tc < N_TCHUNK - 1:
                fire_pos(tc + 1)

        pv = poss[tc % 2]

        def add_row(r, carry, j=j, pv=pv):
            for v in range(VPR):
                sl = pl.ds(v * L, L)
                plsc.addupdate(toks[j].at[r, sl], pv[r, sl])
            return carry

        lax.fori_loop(0, CHUNK, add_row, 0)

        fire_scatter(s)

        s2 = s + LA
        if s2 < NSTEP:
            if s2 >= NBUF:
                wait_scatter(s2 - NBUF)
            fire_gather(s2)

    for s in range(NSTEP - NBUF, NSTEP):
        wait_scatter(s)


@jax.jit
def kernel(idx, tok_table, pos_table):
    idx32 = idx.astype(jnp.int32)
    mesh = plsc.VectorSubcoreMesh(
        core_axis_name="c", subcore_axis_name="s", num_cores=NC,
        num_subcores=NS)

    def body(idx_hbm, pos_hbm, tok_hbm, out_hbm, idx_v, p0, p1,
             t0, t1, t2, t3, g0, g1, g2, g3, s0, s1, s2, s3, ps0, ps1, isem):
        _body(idx_hbm, pos_hbm, tok_hbm, out_hbm, idx_v, [p0, p1],
              [t0, t1, t2, t3], [g0, g1, g2, g3], [s0, s1, s2, s3],
              [ps0, ps1], isem)

    return pl.kernel(
        body,
        out_type=jax.ShapeDtypeStruct((B, T, N_EMBD), jnp.float32),
        mesh=mesh,
        scratch_types=[
            pltpu.VMEM((B * T_PER_W,), jnp.int32),
        ] + [pltpu.VMEM((CHUNK, N_EMBD), jnp.float32)] * 2
          + [pltpu.VMEM((CHUNK, N_EMBD), jnp.float32)] * NBUF
          + [pltpu.SemaphoreType.DMA] * (2 * NBUF + 3),
    )(idx32, pos_table, tok_table)


# speedup vs baseline: 1.4902x; 1.0355x over previous
import jax
import jax.numpy as jnp
from jax import lax
from jax.experimental import pallas as pl
from jax.experimental.pallas import tpu as pltpu
from jax.experimental.pallas import tpu_sc as plsc

VOCAB_SIZE = 32000
N_EMBD = 1024
B, T = 4, 2048

NC, NS, L = 2, 16, 16
NW = NC * NS
T_PER_W = T // NW
CHUNK = 16
N_TCHUNK = T_PER_W // CHUNK
NSTEP = N_TCHUNK * B
NBUF = 5
LA = 3
VPR = N_EMBD // L


def _body(idx_hbm, pos_hbm, tok_hbm, out_hbm, idx_v, poss, toks, gsems, ssems,
          psems, isem):
    w = lax.axis_index("s") * NC + lax.axis_index("c")
    t0 = w * T_PER_W

    def fire_pos(tc):
        pltpu.async_copy(
            pos_hbm.at[pl.ds(t0 + tc * CHUNK, CHUNK)],
            poss[tc % 2],
            psems[tc % 2],
        )

    def wait_pos(tc):
        pltpu.make_async_copy(
            pos_hbm.at[pl.ds(t0 + tc * CHUNK, CHUNK)],
            poss[tc % 2],
            psems[tc % 2],
        ).wait()

    fire_pos(0)
    fire_pos(1)

    for b in range(B):
        pltpu.async_copy(
            idx_hbm.at[b, pl.ds(t0, T_PER_W)],
            idx_v.at[pl.ds(b * T_PER_W, T_PER_W)], isem)
    for b in range(B):
        pltpu.make_async_copy(
            idx_hbm.at[b, pl.ds(t0, T_PER_W)],
            idx_v.at[pl.ds(b * T_PER_W, T_PER_W)], isem).wait()

    def _idx_ref(s):
        tc, b = divmod(s, B)
        return idx_v.at[pl.ds(b * T_PER_W + tc * CHUNK, CHUNK)]

    def fire_gather(s):
        j = s % NBUF
        pltpu.async_copy(tok_hbm.at[_idx_ref(s)], toks[j], gsems[j])

    def wait_gather(s):
        j = s % NBUF
        pltpu.make_async_copy(tok_hbm.at[_idx_ref(s)], toks[j], gsems[j]).wait()

    def fire_scatter(s):
        tc, b = divmod(s, B)
        j = s % NBUF
        pltpu.async_copy(
            toks[j], out_hbm.at[b, pl.ds(t0 + tc * CHUNK, CHUNK)], ssems[j])

    def wait_scatter(s):
        tc, b = divmod(s, B)
        j = s % NBUF
        pltpu.make_async_copy(
            toks[j], out_hbm.at[b, pl.ds(t0 + tc * CHUNK, CHUNK)],
            ssems[j]).wait()

    for s in range(LA):
        fire_gather(s)

    for s in range(NSTEP):
        j = s % NBUF
        tc = s // B
        wait_gather(s)
        if s % B == 0:
            wait_pos(tc)
            if 1 <= tc < N_TCHUNK - 1:
                fire_pos(tc + 1)

        pv = poss[tc % 2]

        def add_row(r, carry, j=j, pv=pv):
            for v in range(VPR):
                sl = pl.ds(v * L, L)
                plsc.addupdate(toks[j].at[r, sl], pv[r, sl])
            return carry

        lax.fori_loop(0, CHUNK, add_row, 0)

        fire_scatter(s)

        s2 = s + LA
        if s2 < NSTEP:
            if s2 >= NBUF:
                wait_scatter(s2 - NBUF)
            fire_gather(s2)

    for s in range(NSTEP - NBUF, NSTEP):
        wait_scatter(s)


@jax.jit
def kernel(idx, tok_table, pos_table):
    idx32 = idx.astype(jnp.int32)
    mesh = plsc.VectorSubcoreMesh(
        core_axis_name="c", subcore_axis_name="s", num_cores=NC,
        num_subcores=NS)

    def body(idx_hbm, pos_hbm, tok_hbm, out_hbm, idx_v, p0, p1,
             t0, t1, t2, t3, t4, g0, g1, g2, g3, g4,
             s0, s1, s2, s3, s4, ps0, ps1, isem):
        _body(idx_hbm, pos_hbm, tok_hbm, out_hbm, idx_v, [p0, p1],
              [t0, t1, t2, t3, t4], [g0, g1, g2, g3, g4],
              [s0, s1, s2, s3, s4], [ps0, ps1], isem)

    return pl.kernel(
        body,
        out_type=jax.ShapeDtypeStruct((B, T, N_EMBD), jnp.float32),
        mesh=mesh,
        scratch_types=[
            pltpu.VMEM((B * T_PER_W,), jnp.int32),
        ] + [pltpu.VMEM((CHUNK, N_EMBD), jnp.float32)] * 2
          + [pltpu.VMEM((CHUNK, N_EMBD), jnp.float32)] * NBUF
          + [pltpu.SemaphoreType.DMA] * (2 * NBUF + 3),
    )(idx32, pos_table, tok_table)


# speedup vs baseline: 1.5325x; 1.0284x over previous
import jax
import jax.numpy as jnp
from jax import lax
from jax.experimental import pallas as pl
from jax.experimental.pallas import tpu as pltpu
from jax.experimental.pallas import tpu_sc as plsc

VOCAB_SIZE = 32000
N_EMBD = 1024
B, T = 4, 2048

NC, NS, L = 2, 16, 16
NW = NC * NS
T_PER_W = T // NW
CHUNK = 16
N_TCHUNK = T_PER_W // CHUNK
NSTEP = N_TCHUNK * B
NBUF = 4
LA = 2
INNER = 2 * B
OUTER = NSTEP // INNER
VPR = N_EMBD // L


def _body(idx_hbm, pos_hbm, tok_hbm, out_hbm, idx_v, poss, toks, gsems, ssems,
          psems, isem):
    w = lax.axis_index("s") * NC + lax.axis_index("c")
    t0 = w * T_PER_W

    def fire_pos(tc, par):
        pltpu.async_copy(
            pos_hbm.at[pl.ds(t0 + tc * CHUNK, CHUNK)],
            poss[par],
            psems[par],
        )

    def wait_pos(tc, par):
        pltpu.make_async_copy(
            pos_hbm.at[pl.ds(t0 + tc * CHUNK, CHUNK)],
            poss[par],
            psems[par],
        ).wait()

    fire_pos(0, 0)
    fire_pos(1, 1)

    for b in range(B):
        pltpu.async_copy(
            idx_hbm.at[b, pl.ds(t0, T_PER_W)],
            idx_v.at[pl.ds(b * T_PER_W, T_PER_W)], isem)
    for b in range(B):
        pltpu.make_async_copy(
            idx_hbm.at[b, pl.ds(t0, T_PER_W)],
            idx_v.at[pl.ds(b * T_PER_W, T_PER_W)], isem).wait()

    def _idx_ref(tc, b):
        return idx_v.at[pl.ds(b * T_PER_W + tc * CHUNK, CHUNK)]

    def fire_gather(tc, b, j):
        pltpu.async_copy(tok_hbm.at[_idx_ref(tc, b)], toks[j], gsems[j])

    def wait_gather(tc, b, j):
        pltpu.make_async_copy(
            tok_hbm.at[_idx_ref(tc, b)], toks[j], gsems[j]).wait()

    def fire_scatter(tc, b, j):
        pltpu.async_copy(
            toks[j], out_hbm.at[b, pl.ds(t0 + tc * CHUNK, CHUNK)], ssems[j])

    def wait_scatter(tc, b, j):
        pltpu.make_async_copy(
            toks[j], out_hbm.at[b, pl.ds(t0 + tc * CHUNK, CHUNK)],
            ssems[j]).wait()

    for s in range(LA):
        fire_gather(s // B, s % B, s % NBUF)

    def outer(k, carry):
        for i in range(INNER):
            j = i % NBUF
            half = i // B
            b = i % B
            tc = 2 * k + half
            wait_gather(tc, b, j)
            if b == 0:
                wait_pos(tc, half)
                if half == 0:
                    pl.when(k >= 1)(lambda tc=tc: fire_pos(tc + 1, 1))
                else:
                    pl.when(k < OUTER - 1)(
                        lambda tc=tc: fire_pos(tc + 1, 0))

            pv = poss[half]

            def add_row(r, carry2, j=j, pv=pv):
                for v in range(VPR):
                    sl = pl.ds(v * L, L)
                    plsc.addupdate(toks[j].at[r, sl], pv[r, sl])
                return carry2

            lax.fori_loop(0, CHUNK, add_row, 0)

            fire_scatter(tc, b, j)

            i2 = i + LA
            b2, j2 = i2 % B, i2 % NBUF
            tc2 = 2 * k + i2 // B if i2 < INNER else 2 * (k + 1)
            tc_prev = tc2 - (NBUF // B)

            def drain_and_fire(tc2=tc2, tc_prev=tc_prev, b2=b2, j2=j2):
                wait_scatter(tc_prev, b2, j2)
                fire_gather(tc2, b2, j2)

            def fire_only(tc2=tc2, b2=b2, j2=j2):
                fire_gather(tc2, b2, j2)

            if i2 < NBUF:
                pl.when(k >= 1)(drain_and_fire)
                pl.when(k == 0)(fire_only)
            elif i2 >= INNER:
                pl.when(k < OUTER - 1)(drain_and_fire)
            else:
                drain_and_fire()
        return carry

    lax.fori_loop(0, OUTER, outer, 0)

    for s in range(NSTEP - NBUF, NSTEP):
        wait_scatter(s // B, s % B, s % NBUF)


@jax.jit
def kernel(idx, tok_table, pos_table):
    idx32 = idx.astype(jnp.int32)
    mesh = plsc.VectorSubcoreMesh(
        core_axis_name="c", subcore_axis_name="s", num_cores=NC,
        num_subcores=NS)

    def body(idx_hbm, pos_hbm, tok_hbm, out_hbm, idx_v, p0, p1,
             t0, t1, t2, t3, g0, g1, g2, g3, s0, s1, s2, s3, ps0, ps1, isem):
        _body(idx_hbm, pos_hbm, tok_hbm, out_hbm, idx_v, [p0, p1],
              [t0, t1, t2, t3], [g0, g1, g2, g3], [s0, s1, s2, s3],
              [ps0, ps1], isem)

    return pl.kernel(
        body,
        out_type=jax.ShapeDtypeStruct((B, T, N_EMBD), jnp.float32),
        mesh=mesh,
        scratch_types=[
            pltpu.VMEM((B * T_PER_W,), jnp.int32),
        ] + [pltpu.VMEM((CHUNK, N_EMBD), jnp.float32)] * 2
          + [pltpu.VMEM((CHUNK, N_EMBD), jnp.float32)] * NBUF
          + [pltpu.SemaphoreType.DMA] * (2 * NBUF + 3),
    )(idx32, pos_table, tok_table)


# speedup vs baseline: 1.5811x; 1.0317x over previous
import jax
import jax.numpy as jnp
from jax import lax
from jax.experimental import pallas as pl
from jax.experimental.pallas import tpu as pltpu
from jax.experimental.pallas import tpu_sc as plsc

VOCAB_SIZE = 32000
N_EMBD = 1024
B, T = 4, 2048

NC, NS, L = 2, 16, 16
NW = NC * NS
T_PER_W = T // NW
CHUNK = 8
N_TCHUNK = T_PER_W // CHUNK
NSTEP = N_TCHUNK * B
NBUF = 8
LA = 4
INNER = 2 * B
OUTER = NSTEP // INNER
VPR = N_EMBD // L


def _body(idx_hbm, pos_hbm, tok_hbm, out_hbm, idx_v, poss, toks, gsems, ssems,
          psems, isem):
    w = lax.axis_index("s") * NC + lax.axis_index("c")
    t0 = w * T_PER_W

    def fire_pos(tc, par):
        pltpu.async_copy(
            pos_hbm.at[pl.ds(t0 + tc * CHUNK, CHUNK)],
            poss[par],
            psems[par],
        )

    def wait_pos(tc, par):
        pltpu.make_async_copy(
            pos_hbm.at[pl.ds(t0 + tc * CHUNK, CHUNK)],
            poss[par],
            psems[par],
        ).wait()

    fire_pos(0, 0)
    fire_pos(1, 1)

    for b in range(B):
        pltpu.async_copy(
            idx_hbm.at[b, pl.ds(t0, T_PER_W)],
            idx_v.at[pl.ds(b * T_PER_W, T_PER_W)], isem)
    for b in range(B):
        pltpu.make_async_copy(
            idx_hbm.at[b, pl.ds(t0, T_PER_W)],
            idx_v.at[pl.ds(b * T_PER_W, T_PER_W)], isem).wait()

    def _idx_ref(tc, b):
        return idx_v.at[pl.ds(b * T_PER_W + tc * CHUNK, CHUNK)]

    def fire_gather(tc, b, j):
        pltpu.async_copy(tok_hbm.at[_idx_ref(tc, b)], toks[j], gsems[j])

    def wait_gather(tc, b, j):
        pltpu.make_async_copy(
            tok_hbm.at[_idx_ref(tc, b)], toks[j], gsems[j]).wait()

    def fire_scatter(tc, b, j):
        pltpu.async_copy(
            toks[j], out_hbm.at[b, pl.ds(t0 + tc * CHUNK, CHUNK)], ssems[j])

    def wait_scatter(tc, b, j):
        pltpu.make_async_copy(
            toks[j], out_hbm.at[b, pl.ds(t0 + tc * CHUNK, CHUNK)],
            ssems[j]).wait()

    for s in range(LA):
        fire_gather(s // B, s % B, s % NBUF)

    def outer(k, carry):
        for i in range(INNER):
            j = i % NBUF
            half = i // B
            b = i % B
            tc = 2 * k + half
            wait_gather(tc, b, j)
            if b == 0:
                wait_pos(tc, half)
                if half == 0:
                    pl.when(k >= 1)(lambda tc=tc: fire_pos(tc + 1, 1))
                else:
                    pl.when(k < OUTER - 1)(
                        lambda tc=tc: fire_pos(tc + 1, 0))

            pv = poss[half]

            def add_row(r, carry2, j=j, pv=pv):
                for v in range(VPR):
                    sl = pl.ds(v * L, L)
                    plsc.addupdate(toks[j].at[r, sl], pv[r, sl])
                return carry2

            lax.fori_loop(0, CHUNK, add_row, 0)

            fire_scatter(tc, b, j)

            i2 = i + LA
            b2, j2 = i2 % B, i2 % NBUF
            tc2 = (2 * k + i2 // B if i2 < INNER
                   else 2 * (k + 1) + (i2 - INNER) // B)
            tc_prev = tc2 - (NBUF // B)

            def drain_and_fire(tc2=tc2, tc_prev=tc_prev, b2=b2, j2=j2):
                wait_scatter(tc_prev, b2, j2)
                fire_gather(tc2, b2, j2)

            def fire_only(tc2=tc2, b2=b2, j2=j2):
                fire_gather(tc2, b2, j2)

            if i2 < NBUF:
                pl.when(k >= 1)(drain_and_fire)
                pl.when(k == 0)(fire_only)
            elif i2 >= INNER:
                pl.when(k < OUTER - 1)(drain_and_fire)
            else:
                drain_and_fire()
        return carry

    lax.fori_loop(0, OUTER, outer, 0)

    for s in range(NSTEP - NBUF, NSTEP):
        wait_scatter(s // B, s % B, s % NBUF)


@jax.jit
def kernel(idx, tok_table, pos_table):
    idx32 = idx.astype(jnp.int32)
    mesh = plsc.VectorSubcoreMesh(
        core_axis_name="c", subcore_axis_name="s", num_cores=NC,
        num_subcores=NS)

    def body(idx_hbm, pos_hbm, tok_hbm, out_hbm, idx_v, p0, p1, *rest):
        toks = list(rest[:NBUF])
        gsems = list(rest[NBUF:2 * NBUF])
        ssems = list(rest[2 * NBUF:3 * NBUF])
        ps0, ps1, isem = rest[3 * NBUF:]
        _body(idx_hbm, pos_hbm, tok_hbm, out_hbm, idx_v, [p0, p1],
              toks, gsems, ssems, [ps0, ps1], isem)

    return pl.kernel(
        body,
        out_type=jax.ShapeDtypeStruct((B, T, N_EMBD), jnp.float32),
        mesh=mesh,
        scratch_types=[
            pltpu.VMEM((B * T_PER_W,), jnp.int32),
        ] + [pltpu.VMEM((CHUNK, N_EMBD), jnp.float32)] * 2
          + [pltpu.VMEM((CHUNK, N_EMBD), jnp.float32)] * NBUF
          + [pltpu.SemaphoreType.DMA] * (2 * NBUF + 3),
    )(idx32, pos_table, tok_table)
